# baseline (speedup 1.0000x reference)
import jax
import jax.numpy as jnp
from jax.experimental import pallas as pl
from jax.experimental.pallas import tpu as pltpu

_NUM_PCS = 5
_PCS_RANGE = 2.0
_SPACING = 2.0 * _PCS_RANGE / _NUM_PCS
_INV_SPACING = 1.0 / _SPACING


def _plf_kernel(p_ref, c_ref, o_ref):
    p = p_ref[...]
    t = p * _INV_SPACING + (_NUM_PCS / 2)
    left = jnp.clip(jnp.floor(t), 0.0, _NUM_PCS - 1.0)
    w = t - left
    c = [c_ref[k] for k in range(_NUM_PCS + 1)]
    m = [left == k for k in range(_NUM_PCS - 1)]
    cl = c[_NUM_PCS - 1]
    cr = c[_NUM_PCS]
    for k in range(_NUM_PCS - 2, -1, -1):
        cl = jnp.where(m[k], c[k], cl)
        cr = jnp.where(m[k], c[k + 1], cr)
    o_ref[...] = (1.0 - w) * cl + w * cr


def _plf_call(param, ctrl, block_rows):
    rows, cols = param.shape
    ctrl_t = jnp.transpose(ctrl, (2, 0, 1))
    grid = (rows // block_rows,)
    return pl.pallas_call(
        _plf_kernel,
        grid=grid,
        in_specs=[
            pl.BlockSpec((block_rows, cols), lambda i: (i, 0)),
            pl.BlockSpec((_NUM_PCS + 1, block_rows, cols), lambda i: (0, i, 0)),
        ],
        out_specs=pl.BlockSpec((block_rows, cols), lambda i: (i, 0)),
        out_shape=jax.ShapeDtypeStruct((rows, cols), param.dtype),
        compiler_params=pltpu.CompilerParams(
            dimension_semantics=("parallel",),
            vmem_limit_bytes=100 * 1024 * 1024,
        ),
    )(param, ctrl_t)


def kernel(param1, param2, ctrl1, ctrl2):
    return (
        _plf_call(param1, ctrl1, 128),
        _plf_call(param2, ctrl2, 40),
    )


# speedup vs baseline: 1.0085x; 1.0085x over previous
import jax
import jax.numpy as jnp
from jax.experimental import pallas as pl
from jax.experimental.pallas import tpu as pltpu

_NUM_PCS = 5
_PCS_RANGE = 2.0
_SPACING = 2.0 * _PCS_RANGE / _NUM_PCS
_INV_SPACING = 1.0 / _SPACING


def _plf_kernel(p_ref, c_ref, o_ref):
    p = p_ref[...]
    t = p * _INV_SPACING + (_NUM_PCS / 2)
    left = jnp.clip(jnp.floor(t), 0.0, _NUM_PCS - 1.0)
    w = t - left
    c = [c_ref[k] for k in range(_NUM_PCS + 1)]
    m = [left == k for k in range(_NUM_PCS - 1)]
    cl = c[_NUM_PCS - 1]
    cr = c[_NUM_PCS]
    for k in range(_NUM_PCS - 2, -1, -1):
        cl = jnp.where(m[k], c[k], cl)
        cr = jnp.where(m[k], c[k + 1], cr)
    o_ref[...] = (1.0 - w) * cl + w * cr


def _plf_call(param, ctrl, block_rows, block_cols=None):
    rows, cols = param.shape
    bc = cols if block_cols is None else block_cols
    ctrl_t = jnp.transpose(ctrl, (2, 0, 1))
    grid = (rows // block_rows, cols // bc)
    return pl.pallas_call(
        _plf_kernel,
        grid=grid,
        in_specs=[
            pl.BlockSpec((block_rows, bc), lambda i, j: (i, j)),
            pl.BlockSpec((_NUM_PCS + 1, block_rows, bc), lambda i, j: (0, i, j)),
        ],
        out_specs=pl.BlockSpec((block_rows, bc), lambda i, j: (i, j)),
        out_shape=jax.ShapeDtypeStruct((rows, cols), param.dtype),
        compiler_params=pltpu.CompilerParams(
            dimension_semantics=("parallel", "arbitrary"),
            vmem_limit_bytes=100 * 1024 * 1024,
        ),
    )(param, ctrl_t)


def kernel(param1, param2, ctrl1, ctrl2):
    return (
        _plf_call(param1, ctrl1, 128),
        _plf_call(param2, ctrl2, 200, 2048),
    )


# speedup vs baseline: 1.0090x; 1.0005x over previous
import jax
import jax.numpy as jnp
from jax.experimental import pallas as pl
from jax.experimental.pallas import tpu as pltpu

_NUM_PCS = 5
_PCS_RANGE = 2.0
_SPACING = 2.0 * _PCS_RANGE / _NUM_PCS
_INV_SPACING = 1.0 / _SPACING


def _plf_kernel(p_ref, c_ref, o_ref):
    p = p_ref[...]
    t = p * _INV_SPACING + (_NUM_PCS / 2)
    left = jnp.clip(jnp.floor(t), 0.0, _NUM_PCS - 1.0)
    w = t - left
    c = [c_ref[k] for k in range(_NUM_PCS + 1)]
    m = [left == k for k in range(_NUM_PCS - 1)]
    cl = c[_NUM_PCS - 1]
    cr = c[_NUM_PCS]
    for k in range(_NUM_PCS - 2, -1, -1):
        cl = jnp.where(m[k], c[k], cl)
        cr = jnp.where(m[k], c[k + 1], cr)
    o_ref[...] = (1.0 - w) * cl + w * cr


def _plf_call(param, ctrl, block_rows, block_cols=None, col_parallel=False):
    rows, cols = param.shape
    bc = cols if block_cols is None else block_cols
    ctrl_t = jnp.transpose(ctrl, (2, 0, 1))
    if col_parallel:
        grid = (cols // bc, rows // block_rows)
        pmap = lambda j, i: (i, j)
        cmap = lambda j, i: (0, i, j)
    else:
        grid = (rows // block_rows, cols // bc)
        pmap = lambda i, j: (i, j)
        cmap = lambda i, j: (0, i, j)
    return pl.pallas_call(
        _plf_kernel,
        grid=grid,
        in_specs=[
            pl.BlockSpec((block_rows, bc), pmap),
            pl.BlockSpec((_NUM_PCS + 1, block_rows, bc), cmap),
        ],
        out_specs=pl.BlockSpec((block_rows, bc), pmap),
        out_shape=jax.ShapeDtypeStruct((rows, cols), param.dtype),
        compiler_params=pltpu.CompilerParams(
            dimension_semantics=("parallel", "arbitrary"),
            vmem_limit_bytes=100 * 1024 * 1024,
        ),
    )(param, ctrl_t)


def kernel(param1, param2, ctrl1, ctrl2):
    return (
        _plf_call(param1, ctrl1, 128),
        _plf_call(param2, ctrl2, 200, 2048, col_parallel=True),
    )
